# Initial kernel scaffold; baseline (speedup 1.0000x reference)
import jax
import jax.numpy as jnp
from jax.experimental import pallas as pl


def kernel(src_logits, hoi_text_similarity, target_classes_i):
    raise NotImplementedError("write your pallas kernel here")


# speedup vs baseline: 1.1387x; 1.1387x over previous
import functools

import jax
import jax.numpy as jnp
from jax import lax
from jax.experimental import pallas as pl
from jax.experimental.pallas import tpu as pltpu
from jax.experimental.pallas import tpu_sc as plsc

_LANES = 16
_CHUNK = 128


def _sc_gather_fn(N, C, NC, NS):
    NW = NC * NS
    bpw = N // NW
    nchunk = bpw // _CHUNK

    mesh = plsc.VectorSubcoreMesh(core_axis_name="c", subcore_axis_name="s")

    @functools.partial(
        pl.kernel,
        out_type=[
            jax.ShapeDtypeStruct((N // _CHUNK, _CHUNK), jnp.float32),
            jax.ShapeDtypeStruct((N // _CHUNK, _CHUNK), jnp.float32),
        ],
        mesh=mesh,
        scratch_types=[
            pltpu.VMEM((bpw,), jnp.int32),
            pltpu.VMEM((nchunk, _CHUNK), jnp.int32),
            pltpu.VMEM((nchunk, _CHUNK), jnp.float32),
            pltpu.VMEM((nchunk, _CHUNK), jnp.float32),
            pltpu.SemaphoreType.DMA,
        ],
    )
    def sc_gather(t_hbm, sim_hbm, log_hbm, simt_out, logt_out,
                  t_v, idx_v, sim_v, log_v, sem):
        wid = lax.axis_index("s") * NC + lax.axis_index("c")
        base = wid * bpw
        pltpu.sync_copy(t_hbm.at[pl.ds(base, bpw)], t_v)
        for j in range(bpw // _LANES):
            tv = t_v[pl.ds(j * _LANES, _LANES)]
            rows = lax.iota(jnp.int32, _LANES) + (base + j * _LANES)
            idx_v[j // (_CHUNK // _LANES),
                  pl.ds((j % (_CHUNK // _LANES)) * _LANES, _LANES)] = (
                rows * C + tv)
        copies = []
        for c in range(nchunk):
            copies.append(pltpu.async_copy(sim_hbm.at[idx_v.at[c]],
                                           sim_v.at[c], sem))
            copies.append(pltpu.async_copy(log_hbm.at[idx_v.at[c]],
                                           log_v.at[c], sem))
        for cp in copies:
            cp.wait()
        pltpu.sync_copy(sim_v, simt_out.at[pl.ds(wid * nchunk, nchunk)])
        pltpu.sync_copy(log_v, logt_out.at[pl.ds(wid * nchunk, nchunk)])

    return sc_gather


def _tc_loss_fn(N, C, R):
    def tc_body(x_ref, simt_ref, logt_ref, out_ref):
        i = pl.program_id(0)

        @pl.when(i == 0)
        def _init():
            out_ref[0, 0] = 0.0

        x = x_ref[...]
        m = jnp.max(x, axis=1, keepdims=True)
        s = jnp.sum(jnp.exp(x - m), axis=1, keepdims=True)
        lse = m + jnp.log(s)
        w = 10.0 / (1.0 + jnp.exp(4.0 * simt_ref[...]))
        out_ref[0, 0] += jnp.sum(w * (lse - logt_ref[...]))

    return pl.pallas_call(
        tc_body,
        grid=(N // R,),
        in_specs=[
            pl.BlockSpec((R, C), lambda i: (i, 0)),
            pl.BlockSpec((R, 1), lambda i: (i, 0)),
            pl.BlockSpec((R, 1), lambda i: (i, 0)),
        ],
        out_specs=pl.BlockSpec(memory_space=pltpu.MemorySpace.SMEM),
        out_shape=jax.ShapeDtypeStruct((1, 1), jnp.float32),
        compiler_params=pltpu.CompilerParams(
            dimension_semantics=("arbitrary",)),
    )


def kernel(src_logits, hoi_text_similarity, target_classes_i):
    N, C = src_logits.shape
    t = target_classes_i.astype(jnp.int32)

    info = plsc.get_sparse_core_info()
    NC, NS = info.num_cores, info.num_subcores

    simt, logt = _sc_gather_fn(N, C, NC, NS)(
        t, hoi_text_similarity.reshape(-1), src_logits.reshape(-1))

    R = 1024
    out = _tc_loss_fn(N, C, R)(
        src_logits, simt.reshape(N, 1), logt.reshape(N, 1))
    return out[0, 0] / N


# speedup vs baseline: 2.0597x; 1.8087x over previous
import jax
import jax.numpy as jnp
from jax import lax
from jax.experimental import pallas as pl
from jax.experimental.pallas import tpu as pltpu

_G = 128


def _tc_loss_fn(S, C, R):
    def body(x_ref, s_ref, t_ref, out_ref):
        i = pl.program_id(0)

        @pl.when(i == 0)
        def _init():
            out_ref[0, 0] = 0.0

        cols = lax.broadcasted_iota(jnp.int32, (R, _G, C), 2)
        oh = cols == t_ref[...][:, :, None]
        x = x_ref[...]
        m = jnp.max(x, axis=2)
        s = jnp.sum(jnp.exp(x - m[:, :, None]), axis=2)
        lse = m + jnp.log(s)
        logit_t = jnp.sum(jnp.where(oh, x, 0.0), axis=2)
        sim_t = jnp.sum(jnp.where(oh, s_ref[...], 0.0), axis=2)
        w = 10.0 / (1.0 + jnp.exp(4.0 * sim_t))
        out_ref[0, 0] += jnp.sum(w * (lse - logit_t))

    return pl.pallas_call(
        body,
        grid=(S // R,),
        in_specs=[
            pl.BlockSpec((R, _G, C), lambda i: (i, 0, 0)),
            pl.BlockSpec((R, _G, C), lambda i: (i, 0, 0)),
            pl.BlockSpec((R, _G), lambda i: (i, 0)),
        ],
        out_specs=pl.BlockSpec(memory_space=pltpu.MemorySpace.SMEM),
        out_shape=jax.ShapeDtypeStruct((1, 1), jnp.float32),
        compiler_params=pltpu.CompilerParams(
            dimension_semantics=("arbitrary",)),
    )


def kernel(src_logits, hoi_text_similarity, target_classes_i):
    N, C = src_logits.shape
    S = N // _G
    x3 = src_logits.reshape(S, _G, C)
    s3 = hoi_text_similarity.reshape(S, _G, C)
    t2 = target_classes_i.astype(jnp.int32).reshape(S, _G)
    R = 8
    out = _tc_loss_fn(S, C, R)(x3, s3, t2)
    return out[0, 0] / N


# speedup vs baseline: 2.1277x; 1.0330x over previous
import jax
import jax.numpy as jnp
from jax import lax
from jax.experimental import pallas as pl
from jax.experimental.pallas import tpu as pltpu

_G = 128


def _tc_loss_fn(S, C, R):
    def body(x_ref, s_ref, t_ref, out_ref):
        i = pl.program_id(0)

        @pl.when(i == 0)
        def _init():
            out_ref[0, 0] = 0.0

        cols = lax.broadcasted_iota(jnp.int32, (R, _G, C), 2)
        oh = cols == t_ref[...][:, :, None]
        x = x_ref[...]
        m = jnp.max(x, axis=2)
        s = jnp.sum(jnp.exp(x - m[:, :, None]), axis=2)
        lse = m + jnp.log(s)
        logit_t = jnp.sum(jnp.where(oh, x, 0.0), axis=2)
        sim_t = jnp.sum(jnp.where(oh, s_ref[...], 0.0), axis=2)
        w = 10.0 / (1.0 + jnp.exp(4.0 * sim_t))
        out_ref[0, 0] += jnp.sum(w * (lse - logit_t))

    return pl.pallas_call(
        body,
        grid=(S // R,),
        in_specs=[
            pl.BlockSpec((R, _G, C), lambda i: (i, 0, 0)),
            pl.BlockSpec((R, _G, C), lambda i: (i, 0, 0)),
            pl.BlockSpec((R, _G), lambda i: (i, 0)),
        ],
        out_specs=pl.BlockSpec(memory_space=pltpu.MemorySpace.SMEM),
        out_shape=jax.ShapeDtypeStruct((1, 1), jnp.float32),
        compiler_params=pltpu.CompilerParams(
            dimension_semantics=("arbitrary",)),
    )


def kernel(src_logits, hoi_text_similarity, target_classes_i):
    N, C = src_logits.shape
    S = N // _G
    x3 = src_logits.reshape(S, _G, C)
    s3 = hoi_text_similarity.reshape(S, _G, C)
    t2 = target_classes_i.astype(jnp.int32).reshape(S, _G)
    R = 16
    out = _tc_loss_fn(S, C, R)(x3, s3, t2)
    return out[0, 0] / N
